# baseline (speedup 1.0000x reference)
import jax
import jax.numpy as jnp
from jax import lax
from jax.experimental import pallas as pl
from jax.experimental.pallas import tpu as pltpu
from jax.experimental.pallas import tpu_sc as plsc

_E = 6400000
_C = 128
_R = _E // _C
_GRID = 50
_BR = _R // _GRID
_CH = 100
_IR = 2 * _R
_IB = _IR // _GRID

_R0 = (13, 15, 26, 6)
_R1 = (17, 29, 16, 24)
_KS1 = 42
_KS2 = 0x1BD11BDA ^ 42


def _rotl(x, r):
    return (x << jnp.uint32(r)) | (x >> jnp.uint32(32 - r))


def _round(x0, x1, r):
    x0 = x0 + x1
    x1 = x0 ^ _rotl(x1, r)
    return x0, x1


def _threefry_bits(x):
    ks1 = jnp.uint32(_KS1)
    ks2 = jnp.uint32(_KS2)
    x1 = x + ks1
    x0 = x1
    x1 = x0 ^ _rotl(x1, _R0[0])
    for r in _R0[1:]:
        x0, x1 = _round(x0, x1, r)
    x0 = x0 + ks1
    x1 = x1 + jnp.uint32((_KS2 + 1) & 0xFFFFFFFF)
    for r in _R1:
        x0, x1 = _round(x0, x1, r)
    x0 = x0 + ks2
    x1 = x1 + jnp.uint32(2)
    for r in _R0:
        x0, x1 = _round(x0, x1, r)
    x1 = x1 + jnp.uint32(_KS1 + 3)
    for r in _R1:
        x0, x1 = _round(x0, x1, r)
    x0 = x0 + ks1
    x1 = x1 + jnp.uint32((_KS2 + 4) & 0xFFFFFFFF)
    for r in _R0:
        x0, x1 = _round(x0, x1, r)
    x0 = x0 + ks2
    x1 = x1 + jnp.uint32(5)
    return x0 ^ x1


_NC = 2
_NS = 16
_NW = _NC * _NS
_TILES = _E // 128
_TPW = _TILES // _NW
_REM = _TILES - _TPW * _NW


def _sc_copy_body(idx_hbm, oidx_hbm):
    wid = lax.axis_index("s") * _NC + lax.axis_index("c")
    base = pl.multiple_of(wid * (_TPW * 128), 128)
    pltpu.sync_copy(idx_hbm.at[:, pl.ds(base, _TPW * 128)],
                    oidx_hbm.at[:, pl.ds(base, _TPW * 128)])

    @pl.when(wid < _REM)
    def _():
        rb = pl.multiple_of((_TPW * _NW + wid) * 128, 128)
        pltpu.sync_copy(idx_hbm.at[:, pl.ds(rb, 128)],
                        oidx_hbm.at[:, pl.ds(rb, 128)])


def _sc_idx_copy(adj_indices):
    mesh = plsc.VectorSubcoreMesh(core_axis_name="c", subcore_axis_name="s")
    return pl.kernel(
        _sc_copy_body,
        out_type=jax.ShapeDtypeStruct((2, _E), jnp.int32),
        mesh=mesh,
    )(adj_indices)


_IBC = _E // _GRID


def _body(m_ref, inv_ref, vals_ref, idx_ref, ovals_ref, oidx_ref):
    g = pl.program_id(0)
    oidx_ref[...] = idx_ref[...]
    mstar9 = m_ref[0].astype(jnp.uint32) << jnp.uint32(9)
    inv = inv_ref[0]
    row = lax.broadcasted_iota(jnp.uint32, (_CH, _C), 0)
    col = lax.broadcasted_iota(jnp.uint32, (_CH, _C), 1)
    lin = (row << jnp.uint32(7)) + col

    gbase = (g * (_BR * _C)).astype(jnp.uint32)
    for k in range(_BR // _CH):
        bits = _threefry_bits(lin + (gbase + jnp.uint32(k * _CH * _C)))
        keep = bits >= mstar9
        v = vals_ref[k * _CH:(k + 1) * _CH, :]
        ovals_ref[k * _CH:(k + 1) * _CH, :] = jnp.where(keep, v * inv, 0.0)


def kernel(adj_indices, adj_values, keepRate):
    assert adj_values.shape == (_E,) and adj_indices.shape == (2, _E)
    kr = jnp.asarray(keepRate, jnp.float32)
    inv = (1.0 / kr).reshape(1)
    m0 = jnp.ceil((1.0 - kr) * jnp.float32(1 << 23)).astype(jnp.int32)
    cands = jnp.clip(m0 + jnp.arange(-2, 3, dtype=jnp.int32), 0, 1 << 23)
    passing = (cands.astype(jnp.float32) * jnp.float32(2.0 ** -23) + kr) >= 1.0
    mstar = jnp.min(jnp.where(passing, cands, 1 << 23)).reshape(1)

    vals2 = adj_values.reshape(_R, _C)
    ovals, oidx = pl.pallas_call(
        _body,
        grid=(_GRID,),
        in_specs=[
            pl.BlockSpec(memory_space=pltpu.SMEM),
            pl.BlockSpec(memory_space=pltpu.SMEM),
            pl.BlockSpec((_BR, _C), lambda g: (g, 0)),
            pl.BlockSpec((2, _IBC), lambda g: (0, g)),
        ],
        out_specs=[
            pl.BlockSpec((_BR, _C), lambda g: (g, 0)),
            pl.BlockSpec((2, _IBC), lambda g: (0, g)),
        ],
        out_shape=[
            jax.ShapeDtypeStruct((_R, _C), jnp.float32),
            jax.ShapeDtypeStruct((2, _E), jnp.int32),
        ],
        compiler_params=pltpu.CompilerParams(
            dimension_semantics=("parallel",),
        ),
    )(mstar, inv, vals2, adj_indices)
    return oidx, ovals.reshape(_E)


# speedup vs baseline: 1.0591x; 1.0591x over previous
import jax
import jax.numpy as jnp
from jax import lax
from jax.experimental import pallas as pl
from jax.experimental.pallas import tpu as pltpu
from jax.experimental.pallas import tpu_sc as plsc

_E = 6400000
_C = 128
_R = _E // _C
_GRID = 25
_BR = _R // _GRID
_CH = 80
_IR = 2 * _R
_IB = _IR // _GRID

_R0 = (13, 15, 26, 6)
_R1 = (17, 29, 16, 24)
_KS1 = 42
_KS2 = 0x1BD11BDA ^ 42


def _rotl(x, r):
    return (x << jnp.uint32(r)) | (x >> jnp.uint32(32 - r))


def _round(x0, x1, r):
    x0 = x0 + x1
    x1 = x0 ^ _rotl(x1, r)
    return x0, x1


def _threefry_bits(x):
    ks1 = jnp.uint32(_KS1)
    ks2 = jnp.uint32(_KS2)
    x1 = x + ks1
    x0 = x1
    x1 = x0 ^ _rotl(x1, _R0[0])
    for r in _R0[1:]:
        x0, x1 = _round(x0, x1, r)
    x0 = x0 + ks1
    x1 = x1 + jnp.uint32((_KS2 + 1) & 0xFFFFFFFF)
    for r in _R1:
        x0, x1 = _round(x0, x1, r)
    x0 = x0 + ks2
    x1 = x1 + jnp.uint32(2)
    for r in _R0:
        x0, x1 = _round(x0, x1, r)
    x1 = x1 + jnp.uint32(_KS1 + 3)
    for r in _R1:
        x0, x1 = _round(x0, x1, r)
    x0 = x0 + ks1
    x1 = x1 + jnp.uint32((_KS2 + 4) & 0xFFFFFFFF)
    for r in _R0:
        x0, x1 = _round(x0, x1, r)
    x0 = x0 + ks2
    x1 = x1 + jnp.uint32(5)
    return x0 ^ x1


_NC = 2
_NS = 16
_NW = _NC * _NS
_TILES = _E // 128
_TPW = _TILES // _NW
_REM = _TILES - _TPW * _NW


def _sc_copy_body(idx_hbm, oidx_hbm):
    wid = lax.axis_index("s") * _NC + lax.axis_index("c")
    base = pl.multiple_of(wid * (_TPW * 128), 128)
    pltpu.sync_copy(idx_hbm.at[:, pl.ds(base, _TPW * 128)],
                    oidx_hbm.at[:, pl.ds(base, _TPW * 128)])

    @pl.when(wid < _REM)
    def _():
        rb = pl.multiple_of((_TPW * _NW + wid) * 128, 128)
        pltpu.sync_copy(idx_hbm.at[:, pl.ds(rb, 128)],
                        oidx_hbm.at[:, pl.ds(rb, 128)])


def _sc_idx_copy(adj_indices):
    mesh = plsc.VectorSubcoreMesh(core_axis_name="c", subcore_axis_name="s")
    return pl.kernel(
        _sc_copy_body,
        out_type=jax.ShapeDtypeStruct((2, _E), jnp.int32),
        mesh=mesh,
    )(adj_indices)


_IBC = _E // _GRID


def _body(m_ref, inv_ref, vals_ref, idx_ref, ovals_ref, oidx_ref):
    g = pl.program_id(0)
    oidx_ref[...] = idx_ref[...]
    mstar9 = m_ref[0].astype(jnp.uint32) << jnp.uint32(9)
    inv = inv_ref[0]
    row = lax.broadcasted_iota(jnp.uint32, (_CH, _C), 0)
    col = lax.broadcasted_iota(jnp.uint32, (_CH, _C), 1)
    lin = (row << jnp.uint32(7)) + col

    gbase = (g * (_BR * _C)).astype(jnp.uint32)
    for k in range(_BR // _CH):
        bits = _threefry_bits(lin + (gbase + jnp.uint32(k * _CH * _C)))
        keep = bits >= mstar9
        v = vals_ref[k * _CH:(k + 1) * _CH, :]
        ovals_ref[k * _CH:(k + 1) * _CH, :] = jnp.where(keep, v * inv, 0.0)


def kernel(adj_indices, adj_values, keepRate):
    assert adj_values.shape == (_E,) and adj_indices.shape == (2, _E)
    kr = jnp.asarray(keepRate, jnp.float32)
    inv = (1.0 / kr).reshape(1)
    m0 = jnp.ceil((1.0 - kr) * jnp.float32(1 << 23)).astype(jnp.int32)
    cands = jnp.clip(m0 + jnp.arange(-2, 3, dtype=jnp.int32), 0, 1 << 23)
    passing = (cands.astype(jnp.float32) * jnp.float32(2.0 ** -23) + kr) >= 1.0
    mstar = jnp.min(jnp.where(passing, cands, 1 << 23)).reshape(1)

    vals2 = adj_values.reshape(_R, _C)
    ovals, oidx = pl.pallas_call(
        _body,
        grid=(_GRID,),
        in_specs=[
            pl.BlockSpec(memory_space=pltpu.SMEM),
            pl.BlockSpec(memory_space=pltpu.SMEM),
            pl.BlockSpec((_BR, _C), lambda g: (g, 0)),
            pl.BlockSpec((2, _IBC), lambda g: (0, g)),
        ],
        out_specs=[
            pl.BlockSpec((_BR, _C), lambda g: (g, 0)),
            pl.BlockSpec((2, _IBC), lambda g: (0, g)),
        ],
        out_shape=[
            jax.ShapeDtypeStruct((_R, _C), jnp.float32),
            jax.ShapeDtypeStruct((2, _E), jnp.int32),
        ],
        compiler_params=pltpu.CompilerParams(
            dimension_semantics=("arbitrary",),
        ),
    )(mstar, inv, vals2, adj_indices)
    return oidx, ovals.reshape(_E)
